# baseline (speedup 1.0000x reference)
import functools

import jax
import jax.numpy as jnp
from jax import lax
from jax.experimental import pallas as pl
from jax.experimental.pallas import tpu as pltpu
from jax.experimental.pallas import tpu_sc as plsc

_PAD = 32


def _tc_body(n_cls, nb, nj, x_ref, t_ref, cnt_out, fn_out, ces_out,
             cnt_ref, fn_ref, ces_ref):
    b = pl.program_id(0)
    j = pl.program_id(1)

    @pl.when(jnp.logical_and(b == 0, j == 0))
    def _init():
        cnt_ref[...] = jnp.zeros_like(cnt_ref)
        fn_ref[...] = jnp.zeros_like(fn_ref)
        ces_ref[...] = jnp.zeros_like(ces_ref)

    x = x_ref[0]
    t = t_ref[...]

    m = jnp.max(x, axis=0, keepdims=True)
    e = jnp.exp(x - m)
    s = jnp.sum(e, axis=0, keepdims=True)
    lse = m + jnp.log(s)

    cls = jax.lax.broadcasted_iota(jnp.int32, x.shape, 0)
    ohf = (cls == t).astype(jnp.float32)
    xt = jnp.sum(x * ohf, axis=0, keepdims=True)
    ce = lse - xt

    wrongf = (xt < m).astype(jnp.float32)

    cnt_ref[0:n_cls] += jnp.sum(ohf, axis=(1, 2), keepdims=True)
    fn_ref[0:n_cls] += jnp.sum(ohf * wrongf, axis=(1, 2), keepdims=True)
    ces_ref[0:n_cls] += jnp.sum(ohf * ce, axis=(1, 2), keepdims=True)

    @pl.when(jnp.logical_and(b == nb - 1, j == nj - 1))
    def _fin():
        cnt_out[...] = jnp.broadcast_to(cnt_ref[...], cnt_out.shape)
        fn_out[...] = jnp.broadcast_to(fn_ref[...], fn_out.shape)
        ces_out[...] = jnp.broadcast_to(ces_ref[...], ces_out.shape)


def _sc_epilogue_body(n_cls, n_total, cnt_hbm, fn_hbm, ces_hbm, out_hbm,
                      cnt_v, fn_v, ces_v, res_v):
    cid = lax.axis_index("c")
    sid = lax.axis_index("s")
    wid = sid * 2 + cid

    @pl.when(wid == 0)
    def _():
        pltpu.sync_copy(cnt_hbm, cnt_v)
        pltpu.sync_copy(fn_hbm, fn_v)
        pltpu.sync_copy(ces_hbm, ces_v)
        one = jnp.ones((16,), jnp.float32)
        loss = jnp.zeros((16,), jnp.float32)
        for c in range(n_cls):
            cnt = cnt_v[c, 0, 0:16]
            fn = fn_v[c, 0, 0:16]
            ces = ces_v[c, 0, 0:16]
            gt_c = jnp.where(cnt > 0.0, cnt, one)
            fn_c = jnp.where(fn > 0.0, fn, one)
            loss = loss + (fn_c / gt_c) * ces
        res_v[...] = loss / jnp.float32(n_total)
        pltpu.sync_copy(res_v, out_hbm)


def kernel(input, target):
    nb, n_cls, h, w = input.shape
    rows = 64
    nj = h // rows
    n_total = nb * h * w

    body = functools.partial(_tc_body, n_cls, nb, nj)
    part_shape = jax.ShapeDtypeStruct((_PAD, 1, 128), jnp.float32)

    cnt_p, fn_p, ces_p = pl.pallas_call(
        body,
        grid=(nb, nj),
        in_specs=[
            pl.BlockSpec((1, n_cls, rows, w), lambda b, j: (b, 0, j, 0)),
            pl.BlockSpec((1, rows, w), lambda b, j: (b, j, 0)),
        ],
        out_specs=[
            pl.BlockSpec((_PAD, 1, 128), lambda b, j: (0, 0, 0)),
            pl.BlockSpec((_PAD, 1, 128), lambda b, j: (0, 0, 0)),
            pl.BlockSpec((_PAD, 1, 128), lambda b, j: (0, 0, 0)),
        ],
        out_shape=[part_shape, part_shape, part_shape],
        scratch_shapes=[
            pltpu.VMEM((_PAD, 1, 1), jnp.float32),
            pltpu.VMEM((_PAD, 1, 1), jnp.float32),
            pltpu.VMEM((_PAD, 1, 1), jnp.float32),
        ],
        compiler_params=pltpu.CompilerParams(
            dimension_semantics=("arbitrary", "arbitrary"),
        ),
    )(input, target)

    sc_fn = pl.kernel(
        functools.partial(_sc_epilogue_body, n_cls, n_total),
        out_type=jax.ShapeDtypeStruct((16,), jnp.float32),
        mesh=plsc.VectorSubcoreMesh(core_axis_name="c", subcore_axis_name="s"),
        scratch_types=[
            pltpu.VMEM((_PAD, 1, 128), jnp.float32),
            pltpu.VMEM((_PAD, 1, 128), jnp.float32),
            pltpu.VMEM((_PAD, 1, 128), jnp.float32),
            pltpu.VMEM((16,), jnp.float32),
        ],
    )
    out = sc_fn(cnt_p, fn_p, ces_p)
    return out[0]
